# baseline (speedup 1.0000x reference)
import functools

import jax
import jax.numpy as jnp
from jax.experimental import pallas as pl
from jax.experimental.pallas import tpu as pltpu

B, SEQ, PRED, E, K = 4096, 512, 96, 64, 8
T = 512
NEG = -1e30


def _moe_body(x_ref, xb_ref, wg_ref, bg_ref, we_ref, be_ref, out_ref,
              loss_ref, dacc, pacc):
    i = pl.program_id(0)
    x = x_ref[...]
    gate = jnp.dot(x, wg_ref[...], preferred_element_type=jnp.float32)
    gate = gate + bg_ref[...]

    cur = gate
    vals = []
    onehots = []
    lane = jax.lax.broadcasted_iota(jnp.int32, (T, E), 1)
    for _ in range(K):
        m = jnp.max(cur, axis=1, keepdims=True)
        idx = jnp.argmax(cur, axis=1)
        oh = lane == idx[:, None]
        vals.append(m)
        onehots.append(oh)
        cur = jnp.where(oh, NEG, cur)
    v = jnp.concatenate(vals, axis=1)
    ev = jnp.exp(v - v[:, 0:1])
    w = ev / jnp.sum(ev, axis=1, keepdims=True)
    g_combine = jnp.zeros((T, E), dtype=jnp.float32)
    for k in range(K):
        g_combine = g_combine + jnp.where(onehots[k], w[:, k:k + 1], 0.0)

    acc0 = jnp.dot(g_combine, be_ref[...], preferred_element_type=jnp.float32)
    xb = xb_ref[...]

    def body_e(e, acc):
        sel = (lane == e).astype(jnp.float32)
        ge = jnp.sum(g_combine * sel, axis=1, keepdims=True)
        y = jnp.dot(xb, we_ref[e], preferred_element_type=jnp.float32)
        return acc + ge * y

    out_ref[...] = jax.lax.fori_loop(0, E, body_e, acc0)

    gm = jnp.max(gate, axis=1, keepdims=True)
    ex = jnp.exp(gate - gm)
    gp = ex / jnp.sum(ex, axis=1, keepdims=True)
    p_part = jnp.sum(gp, axis=0, keepdims=True)
    d_part = jnp.sum(onehots[0].astype(jnp.float32), axis=0, keepdims=True)

    @pl.when(i == 0)
    def _init():
        dacc[...] = jnp.zeros_like(dacc)
        pacc[...] = jnp.zeros_like(pacc)

    dacc[...] += d_part
    pacc[...] += p_part

    @pl.when(i == pl.num_programs(0) - 1)
    def _fin():
        loss_ref[...] = jnp.sum(dacc[...] * pacc[...]).reshape(1, 1) * (E / (B * B))


@jax.jit
def _moe(x, Wg, bg2, We, be):
    xb = x.astype(jnp.bfloat16)
    web = We.astype(jnp.bfloat16)
    out, loss = pl.pallas_call(
        _moe_body,
        grid=(B // T,),
        in_specs=[
            pl.BlockSpec((T, SEQ), lambda i: (i, 0)),
            pl.BlockSpec((T, SEQ), lambda i: (i, 0)),
            pl.BlockSpec((SEQ, E), lambda i: (0, 0)),
            pl.BlockSpec((1, E), lambda i: (0, 0)),
            pl.BlockSpec((E, SEQ, PRED), lambda i: (0, 0, 0)),
            pl.BlockSpec((E, PRED), lambda i: (0, 0)),
        ],
        out_specs=[
            pl.BlockSpec((T, PRED), lambda i: (i, 0)),
            pl.BlockSpec((1, 1), lambda i: (0, 0)),
        ],
        out_shape=[
            jax.ShapeDtypeStruct((B, PRED), jnp.float32),
            jax.ShapeDtypeStruct((1, 1), jnp.float32),
        ],
        scratch_shapes=[
            pltpu.VMEM((1, E), jnp.float32),
            pltpu.VMEM((1, E), jnp.float32),
        ],
    )(x, xb, Wg, bg2, web, be)
    return out, loss[0, 0]


def kernel(x, Wg, bg, We, be):
    return _moe(x, Wg, bg.reshape(1, E), We, be)


# speedup vs baseline: 1.5497x; 1.5497x over previous
import functools

import jax
import jax.numpy as jnp
from jax.experimental import pallas as pl
from jax.experimental.pallas import tpu as pltpu

B, SEQ, PRED, E, K = 4096, 512, 96, 64, 8
PP = 128
EG = 4
NG = E // EG
T = 512
NEG = -1e30


def _moe_body(x_ref, xb_ref, wg_ref, bg_ref, weg_ref, bep_ref, out_ref,
              loss_ref, dacc, pacc):
    i = pl.program_id(0)
    x = x_ref[...]
    gate = jnp.dot(x, wg_ref[...], preferred_element_type=jnp.float32)
    gate = gate + bg_ref[...]

    cur = gate
    vals = []
    onehots = []
    lane = jax.lax.broadcasted_iota(jnp.int32, (T, E), 1)
    for _ in range(K):
        m = jnp.max(cur, axis=1, keepdims=True)
        idx = jnp.argmax(cur, axis=1)
        oh = lane == idx[:, None]
        vals.append(m)
        onehots.append(oh)
        cur = jnp.where(oh, NEG, cur)
    v = jnp.concatenate(vals, axis=1)
    ev = jnp.exp(v - v[:, 0:1])
    w = ev / jnp.sum(ev, axis=1, keepdims=True)
    g_combine = jnp.zeros((T, E), dtype=jnp.float32)
    for k in range(K):
        g_combine = g_combine + jnp.where(onehots[k], w[:, k:k + 1], 0.0)

    acc0 = jnp.dot(g_combine, bep_ref[...], preferred_element_type=jnp.float32)
    xb = xb_ref[...]

    def body_g(g, acc):
        y = jnp.dot(xb, weg_ref[g], preferred_element_type=jnp.float32)
        for q in range(EG):
            e = EG * g + q
            sel = (lane == e).astype(jnp.float32)
            ge = jnp.sum(g_combine * sel, axis=1, keepdims=True)
            acc = acc + ge * y[:, PP * q:PP * (q + 1)]
        return acc

    acc = jax.lax.fori_loop(0, NG, body_g, acc0)
    out_ref[...] = acc[:, :PRED]

    gm = jnp.max(gate, axis=1, keepdims=True)
    ex = jnp.exp(gate - gm)
    gp = ex / jnp.sum(ex, axis=1, keepdims=True)
    p_part = jnp.sum(gp, axis=0, keepdims=True)
    d_part = jnp.sum(onehots[0].astype(jnp.float32), axis=0, keepdims=True)

    @pl.when(i == 0)
    def _init():
        dacc[...] = jnp.zeros_like(dacc)
        pacc[...] = jnp.zeros_like(pacc)

    dacc[...] += d_part
    pacc[...] += p_part

    @pl.when(i == pl.num_programs(0) - 1)
    def _fin():
        loss_ref[...] = jnp.sum(dacc[...] * pacc[...]).reshape(1, 1) * (E / (B * B))


@jax.jit
def _moe(x, Wg, bg2, We, be):
    xb = x.astype(jnp.bfloat16)
    wep = jnp.pad(We, ((0, 0), (0, 0), (0, PP - PRED))).astype(jnp.bfloat16)
    weg = wep.reshape(NG, EG, SEQ, PP).transpose(0, 2, 1, 3).reshape(NG, SEQ, EG * PP)
    bep = jnp.pad(be, ((0, 0), (0, PP - PRED)))
    out, loss = pl.pallas_call(
        _moe_body,
        grid=(B // T,),
        in_specs=[
            pl.BlockSpec((T, SEQ), lambda i: (i, 0)),
            pl.BlockSpec((T, SEQ), lambda i: (i, 0)),
            pl.BlockSpec((SEQ, E), lambda i: (0, 0)),
            pl.BlockSpec((1, E), lambda i: (0, 0)),
            pl.BlockSpec((NG, SEQ, EG * PP), lambda i: (0, 0, 0)),
            pl.BlockSpec((E, PP), lambda i: (0, 0)),
        ],
        out_specs=[
            pl.BlockSpec((T, PRED), lambda i: (i, 0)),
            pl.BlockSpec((1, 1), lambda i: (0, 0)),
        ],
        out_shape=[
            jax.ShapeDtypeStruct((B, PRED), jnp.float32),
            jax.ShapeDtypeStruct((1, 1), jnp.float32),
        ],
        scratch_shapes=[
            pltpu.VMEM((1, E), jnp.float32),
            pltpu.VMEM((1, E), jnp.float32),
        ],
    )(x, xb, Wg, bg2, weg, bep)
    return out, loss[0, 0]


def kernel(x, Wg, bg, We, be):
    return _moe(x, Wg, bg.reshape(1, E), We, be)


# speedup vs baseline: 1.7406x; 1.1232x over previous
import functools

import jax
import jax.numpy as jnp
from jax.experimental import pallas as pl
from jax.experimental.pallas import tpu as pltpu

B, SEQ, PRED, E, K = 4096, 512, 96, 64, 8
PP = 128
EG = 4
NG = E // EG
T = 512
NEG = -1e30


def _moe_body(x_ref, xb_ref, wg_ref, bg_ref, weg_ref, bep_ref, out_ref,
              loss_ref, dacc, pacc):
    i = pl.program_id(0)
    x = x_ref[...]
    gate = jnp.dot(x, wg_ref[...], preferred_element_type=jnp.float32)
    gate = gate + bg_ref[...]

    gate_t = gate.T
    cur = gate_t
    vals = []
    onehots = []
    for _ in range(K):
        m = jnp.max(cur, axis=0, keepdims=True)
        oh = cur == m
        vals.append(m)
        onehots.append(oh)
        cur = jnp.where(oh, NEG, cur)
    v = jnp.concatenate(vals, axis=0)
    ev = jnp.exp(v - v[0:1, :])
    w = ev / jnp.sum(ev, axis=0, keepdims=True)
    g_t = jnp.zeros((E, T), dtype=jnp.float32)
    for k in range(K):
        g_t = g_t + jnp.where(onehots[k], w[k:k + 1, :], 0.0)
    g_combine = g_t.T
    lane = jax.lax.broadcasted_iota(jnp.int32, (T, E), 1)

    acc0 = jnp.dot(g_combine, bep_ref[...], preferred_element_type=jnp.float32)
    xb = xb_ref[...]

    def body_g(g, acc):
        y = jnp.dot(xb, weg_ref[g], preferred_element_type=jnp.float32)
        for q in range(EG):
            e = EG * g + q
            sel = (lane == e).astype(jnp.float32)
            ge = jnp.sum(g_combine * sel, axis=1, keepdims=True)
            acc = acc + ge * y[:, PP * q:PP * (q + 1)]
        return acc

    acc = jax.lax.fori_loop(0, NG, body_g, acc0)
    out_ref[...] = acc[:, :PRED]

    ex = jnp.exp(gate_t - vals[0])
    gp = ex / jnp.sum(ex, axis=0, keepdims=True)
    p_part = jnp.sum(gp, axis=1, keepdims=True)
    d_part = jnp.sum(onehots[0].astype(jnp.float32), axis=1, keepdims=True)

    @pl.when(i == 0)
    def _init():
        dacc[...] = jnp.zeros_like(dacc)
        pacc[...] = jnp.zeros_like(pacc)

    dacc[...] += d_part
    pacc[...] += p_part

    @pl.when(i == pl.num_programs(0) - 1)
    def _fin():
        loss_ref[...] = jnp.sum(dacc[...] * pacc[...]).reshape(1, 1) * (E / (B * B))


@jax.jit
def _moe(x, Wg, bg2, We, be):
    xb = x.astype(jnp.bfloat16)
    wep = jnp.pad(We, ((0, 0), (0, 0), (0, PP - PRED))).astype(jnp.bfloat16)
    weg = wep.reshape(NG, EG, SEQ, PP).transpose(0, 2, 1, 3).reshape(NG, SEQ, EG * PP)
    bep = jnp.pad(be, ((0, 0), (0, PP - PRED)))
    out, loss = pl.pallas_call(
        _moe_body,
        grid=(B // T,),
        in_specs=[
            pl.BlockSpec((T, SEQ), lambda i: (i, 0)),
            pl.BlockSpec((T, SEQ), lambda i: (i, 0)),
            pl.BlockSpec((SEQ, E), lambda i: (0, 0)),
            pl.BlockSpec((1, E), lambda i: (0, 0)),
            pl.BlockSpec((NG, SEQ, EG * PP), lambda i: (0, 0, 0)),
            pl.BlockSpec((E, PP), lambda i: (0, 0)),
        ],
        out_specs=[
            pl.BlockSpec((T, PRED), lambda i: (i, 0)),
            pl.BlockSpec((1, 1), lambda i: (0, 0)),
        ],
        out_shape=[
            jax.ShapeDtypeStruct((B, PRED), jnp.float32),
            jax.ShapeDtypeStruct((1, 1), jnp.float32),
        ],
        scratch_shapes=[
            pltpu.VMEM((E, 1), jnp.float32),
            pltpu.VMEM((E, 1), jnp.float32),
        ],
    )(x, xb, Wg, bg2, weg, bep)
    return out, loss[0, 0]


def kernel(x, Wg, bg, We, be):
    return _moe(x, Wg, bg.reshape(1, E), We, be)
